# baseline (speedup 1.0000x reference)
import functools

import jax
import jax.numpy as jnp
from jax import lax
from jax.experimental import pallas as pl
from jax.experimental.pallas import tpu as pltpu
from jax.experimental.pallas import tpu_sc as plsc

B, T = 1024, 200
N = B * T
WORD_V, WORD_D = 100000, 128
OTHER_D = 32
POS_V, SUF2_V, SUF3_V, PREF2_V, PREF3_V = 45, 1000, 5000, 1000, 5000
CLS = 45
C = 64
SEG = 3 + WORD_D + 4 * OTHER_D
NSLOT = 5
NFEAT = 27

NC, NS, L = 2, 16, 16
NW = NC * NS
CH = 640
TOK_W = N // NW
CHUNKS = TOK_W // CH
GSL = 128


def _fuse_word_body(x_ref, r_ref, *o_refs):
    res = jnp.dot(x_ref[...], r_ref[...], preferred_element_type=jnp.float32)
    for s, o_ref in enumerate(o_refs):
        o_ref[...] = res[:, s * C:(s + 1) * C].astype(jnp.bfloat16)


def _fuse_word(w_word, rhs):
    M, K = w_word.shape
    BM = 2000
    outs = tuple(jax.ShapeDtypeStruct((M, C), jnp.bfloat16)
                 for _ in range(NSLOT))
    return pl.pallas_call(
        _fuse_word_body,
        grid=(M // BM,),
        in_specs=[pl.BlockSpec((BM, K), lambda i: (i, 0)),
                  pl.BlockSpec((K, NSLOT * C), lambda i: (0, 0))],
        out_specs=tuple(pl.BlockSpec((BM, C), lambda i: (i, 0))
                        for _ in range(NSLOT)),
        out_shape=outs,
    )(w_word, rhs)


def _fuse_small_body(s2, s3, p2, p3, pos, r2, r3, q2, q3, rp,
                     o2, o3, u2, u3, op):
    o2[...] = jnp.dot(s2[...], r2[...], preferred_element_type=jnp.float32).astype(jnp.bfloat16)
    o3[...] = jnp.dot(s3[...], r3[...], preferred_element_type=jnp.float32).astype(jnp.bfloat16)
    u2[...] = jnp.dot(p2[...], q2[...], preferred_element_type=jnp.float32).astype(jnp.bfloat16)
    u3[...] = jnp.dot(p3[...], q3[...], preferred_element_type=jnp.float32).astype(jnp.bfloat16)
    op[...] = jnp.dot(pos[...], rp[...], preferred_element_type=jnp.float32).astype(jnp.bfloat16)


def _fuse_small(w_s2, w_s3, w_p2, w_p3, w_pos_pad, r2, r3, q2, q3, rp):
    outs = (jax.ShapeDtypeStruct((SUF2_V, NSLOT * C), jnp.bfloat16),
            jax.ShapeDtypeStruct((SUF3_V, NSLOT * C), jnp.bfloat16),
            jax.ShapeDtypeStruct((PREF2_V, NSLOT * C), jnp.bfloat16),
            jax.ShapeDtypeStruct((PREF3_V, NSLOT * C), jnp.bfloat16),
            jax.ShapeDtypeStruct((48, 2 * C), jnp.bfloat16))
    return pl.pallas_call(_fuse_small_body, out_shape=outs)(
        w_s2, w_s3, w_p2, w_p3, w_pos_pad, r2, r3, q2, q3, rp)


def _sc_gather_sum(idx_list, tables, feats_spec, partial_in=None):
    nf = len(feats_spec)
    mesh = plsc.VectorSubcoreMesh(core_axis_name="c", subcore_axis_name="s",
                                  num_cores=NC, num_subcores=NS)

    @functools.partial(
        pl.kernel,
        out_type=jax.ShapeDtypeStruct((N, C), jnp.bfloat16),
        mesh=mesh,
        scratch_types=[
            pltpu.VMEM((nf, CH), jnp.int32),
            pltpu.VMEM((CH, C), jnp.bfloat16),
            pltpu.SemaphoreType.DMA,
            pltpu.SemaphoreType.DMA,
        ],
        compiler_params=pltpu.CompilerParams(use_tc_tiling_on_sc=False),
    )
    def body(*refs):
        idxs = refs[:nf]
        tabs = refs[nf:nf + len(tables)]
        if partial_in is None:
            out_hbm, idxb, acc, sem, sem_i = refs[nf + len(tables):]
            part_hbm = None
        else:
            part_hbm, out_hbm, idxb, acc, sem, sem_i = refs[nf + len(tables):]
        wid = lax.axis_index("s") * NC + lax.axis_index("c")

        def do_chunk(cidx, carry):
            row = wid * CHUNKS + cidx
            ih = [pltpu.async_copy(idxs[f].at[row], idxb.at[f], sem_i)
                  for f in range(nf)]
            if partial_in is None:
                zv = jnp.zeros((2 * L,), jnp.bfloat16)

                def zbod(r, _):
                    for cc in range(C // (2 * L)):
                        acc[r, pl.ds(cc * 2 * L, 2 * L)] = zv
                    return 0
                lax.fori_loop(0, CH, zbod, 0)
            else:
                pltpu.sync_copy(part_hbm.at[pl.ds(row * CH, CH)], acc)
            for h in ih:
                h.wait()

            handles = []
            for f in range(nf):
                ti, mult, off = feats_spec[f]
                tab = tabs[ti]
                if (mult, off) != (1, 0):
                    for k in range(CH // L):
                        sl = pl.ds(k * L, L)
                        idxb[f, sl] = idxb[f, sl] * mult + off
                for j in range(CH // GSL):
                    handles.append(pltpu.async_copy(
                        tab.at[idxb.at[f, pl.ds(j * GSL, GSL)]],
                        acc.at[pl.ds(j * GSL, GSL)], sem, add=True))
            for h in handles:
                h.wait()

            pltpu.sync_copy(acc, out_hbm.at[pl.ds(row * CH, CH)])
            return carry

        lax.fori_loop(0, CHUNKS, do_chunk, 0)

    if partial_in is None:
        return body(*idx_list, *tables)
    return body(*idx_list, *tables, partial_in)


def _finish(partial, feats, wf, bp):
    RB = 16
    R = RB * T

    def fbody(p_ref, f0, f1, f2, f3, f4, w_ref, b_ref, o_ref):
        x = p_ref[...].astype(jnp.float32) + b_ref[...]
        for s, f_ref in enumerate((f0, f1, f2, f3, f4)):
            fv = f_ref[...].reshape(R, 3)
            x = x + jnp.dot(fv, w_ref[3 * s:3 * s + 3, :],
                            preferred_element_type=jnp.float32)
        col = lax.broadcasted_iota(jnp.int32, (R, C), 1)
        valid = col < CLS
        xm = jnp.where(valid, x, -jnp.inf)
        m = jnp.max(xm, axis=1, keepdims=True)
        e = jnp.where(valid, jnp.exp(x - m), 0.0)
        lse = jnp.log(jnp.sum(e, axis=1, keepdims=True)) + m
        o_ref[...] = (x - lse)[:, :CLS]

    fspec = pl.BlockSpec((RB, T, 3), lambda i: (i, 0, 0))
    return pl.pallas_call(
        fbody,
        grid=(B // RB,),
        in_specs=[pl.BlockSpec((R, C), lambda i: (i, 0)),
                  fspec, fspec, fspec, fspec, fspec,
                  pl.BlockSpec((15, C), lambda i: (0, 0)),
                  pl.BlockSpec((1, C), lambda i: (0, 0))],
        out_specs=pl.BlockSpec((R, CLS), lambda i: (i, 0)),
        out_shape=jax.ShapeDtypeStruct((N, CLS), jnp.float32),
    )(partial, *feats, wf, bp)


def kernel(words, words_suf2, words_suf3, words_pref2, words_pref3, words_feat, prev_words, prev_words_suf2, prev_words_suf3, prev_words_pref2, prev_words_pref3, prev_words_feat, prev_prev_words, prev_prev_words_suf2, prev_prev_words_suf3, prev_prev_words_pref2, prev_prev_words_pref3, prev_prev_words_feat, next_words, next_words_suf2, next_words_suf3, next_words_pref2, next_words_pref3, next_words_feat, next_next_words, next_next_words_suf2, next_next_words_suf3, next_next_words_pref2, next_next_words_pref3, next_next_words_feat, prev_pos, prev_prev_pos, W_word, W_pos, W_suf2, W_suf3, W_pref2, W_pref3, W_lin, b_lin):
    def seg_rhs(off, width):
        rs = jnp.stack([W_lin[s * SEG + off: s * SEG + off + width, :]
                        for s in range(NSLOT)], axis=1)
        rs = jnp.pad(rs, ((0, 0), (0, 0), (0, C - CLS)))
        return rs.reshape(width, NSLOT * C)

    rhs_w = seg_rhs(0, WORD_D)
    rhs_s2 = seg_rhs(WORD_D, OTHER_D)
    rhs_s3 = seg_rhs(WORD_D + OTHER_D, OTHER_D)
    rhs_p2 = seg_rhs(WORD_D + 2 * OTHER_D, OTHER_D)
    rhs_p3 = seg_rhs(WORD_D + 3 * OTHER_D, OTHER_D)
    rhs_pos = jnp.stack([W_lin[NSLOT * SEG: NSLOT * SEG + OTHER_D],
                         W_lin[NSLOT * SEG + OTHER_D:]], axis=1)
    rhs_pos = jnp.pad(rhs_pos, ((0, 0), (0, 0), (0, C - CLS)))
    rhs_pos = rhs_pos.reshape(OTHER_D, 2 * C)
    wf = jnp.stack([W_lin[s * SEG + WORD_D + 4 * OTHER_D: (s + 1) * SEG]
                    for s in range(NSLOT)], axis=0).reshape(15, CLS)
    wf = jnp.pad(wf, ((0, 0), (0, C - CLS)))
    bp = jnp.pad(b_lin, (0, C - CLS)).reshape(1, C)

    sm = _fuse_small(W_suf2, W_suf3, W_pref2, W_pref3,
                     jnp.pad(W_pos, ((0, 3), (0, 0))),
                     rhs_s2, rhs_s3, rhs_p2, rhs_p3, rhs_pos)
    t_s2 = sm[0].reshape(SUF2_V * NSLOT, C)
    t_s3 = sm[1].reshape(SUF3_V * NSLOT, C)
    t_p2 = sm[2].reshape(PREF2_V * NSLOT, C)
    t_p3 = sm[3].reshape(PREF3_V * NSLOT, C)
    t_pos = sm[4][:POS_V].reshape(POS_V * 2, C)

    wordsets = [
        (words, words_suf2, words_suf3, words_pref2, words_pref3),
        (prev_words, prev_words_suf2, prev_words_suf3, prev_words_pref2, prev_words_pref3),
        (prev_prev_words, prev_prev_words_suf2, prev_prev_words_suf3, prev_prev_words_pref2, prev_prev_words_pref3),
        (next_words, next_words_suf2, next_words_suf3, next_words_pref2, next_words_pref3),
        (next_next_words, next_next_words_suf2, next_next_words_suf3, next_next_words_pref2, next_next_words_pref3),
    ]
    idx_small = [a.reshape(N // CH, CH) for tup in wordsets for a in tup[1:]]
    idx_small += [prev_pos.reshape(N // CH, CH),
                  prev_prev_pos.reshape(N // CH, CH)]
    idx_word = [tup[0].reshape(N // CH, CH) for tup in wordsets]

    smalls = [t_s2, t_s3, t_p2, t_p3, t_pos]
    feats_small = [(i, NSLOT, s) for s in range(NSLOT) for i in range(4)]
    feats_small += [(4, 2, 0), (4, 2, 1)]
    partial1 = _sc_gather_sum(idx_small, smalls, feats_small)

    t_words = _fuse_word(W_word, rhs_w)
    feats_word = [(s, 1, 0) for s in range(NSLOT)]
    partial = _sc_gather_sum(idx_word, list(t_words), feats_word,
                             partial_in=partial1)

    feats = (words_feat, prev_words_feat, prev_prev_words_feat,
             next_words_feat, next_next_words_feat)
    return _finish(partial, feats, wf, bp)


# speedup vs baseline: 1.1785x; 1.1785x over previous
import functools

import jax
import jax.numpy as jnp
from jax import lax
from jax.experimental import pallas as pl
from jax.experimental.pallas import tpu as pltpu
from jax.experimental.pallas import tpu_sc as plsc

B, T = 1024, 200
N = B * T
WORD_V, WORD_D = 100000, 128
OTHER_D = 32
POS_V, SUF2_V, SUF3_V, PREF2_V, PREF3_V = 45, 1000, 5000, 1000, 5000
CLS = 45
C = 64
SEG = 3 + WORD_D + 4 * OTHER_D
NSLOT = 5
NFEAT = 27

NC, NS, L = 2, 16, 16
NW = NC * NS
CH = 640
TOK_W = N // NW
CHUNKS = TOK_W // CH
GSL = 128


def _fuse_word_body(x_ref, r_ref, o_ref):
    o_ref[...] = jnp.dot(x_ref[...], r_ref[...],
                         preferred_element_type=jnp.float32
                         ).astype(jnp.bfloat16)


def _fuse_word(w_word, rhs):
    M, K = w_word.shape
    Ncol = rhs.shape[1]
    BM = 2000
    return pl.pallas_call(
        _fuse_word_body,
        grid=(M // BM,),
        in_specs=[pl.BlockSpec((BM, K), lambda i: (i, 0)),
                  pl.BlockSpec((K, Ncol), lambda i: (0, 0))],
        out_specs=pl.BlockSpec((BM, Ncol), lambda i: (i, 0)),
        out_shape=jax.ShapeDtypeStruct((M, Ncol), jnp.bfloat16),
    )(w_word, rhs)


def _fuse_small_body(s2, s3, p2, p3, pos, r2, r3, q2, q3, rp,
                     o2, o3, u2, u3, op):
    o2[...] = jnp.dot(s2[...], r2[...], preferred_element_type=jnp.float32).astype(jnp.bfloat16)
    o3[...] = jnp.dot(s3[...], r3[...], preferred_element_type=jnp.float32).astype(jnp.bfloat16)
    u2[...] = jnp.dot(p2[...], q2[...], preferred_element_type=jnp.float32).astype(jnp.bfloat16)
    u3[...] = jnp.dot(p3[...], q3[...], preferred_element_type=jnp.float32).astype(jnp.bfloat16)
    op[...] = jnp.dot(pos[...], rp[...], preferred_element_type=jnp.float32).astype(jnp.bfloat16)


def _fuse_small(w_s2, w_s3, w_p2, w_p3, w_pos_pad, r2, r3, q2, q3, rp):
    outs = (jax.ShapeDtypeStruct((SUF2_V, NSLOT * C), jnp.bfloat16),
            jax.ShapeDtypeStruct((SUF3_V, NSLOT * C), jnp.bfloat16),
            jax.ShapeDtypeStruct((PREF2_V, NSLOT * C), jnp.bfloat16),
            jax.ShapeDtypeStruct((PREF3_V, NSLOT * C), jnp.bfloat16),
            jax.ShapeDtypeStruct((48, 2 * C), jnp.bfloat16))
    return pl.pallas_call(_fuse_small_body, out_shape=outs)(
        w_s2, w_s3, w_p2, w_p3, w_pos_pad, r2, r3, q2, q3, rp)


def _sc_gather_sum(idx_list, tables, feats_spec, partial_in=None):
    nf = len(feats_spec)
    mesh = plsc.VectorSubcoreMesh(core_axis_name="c", subcore_axis_name="s",
                                  num_cores=NC, num_subcores=NS)

    @functools.partial(
        pl.kernel,
        out_type=jax.ShapeDtypeStruct((N, C), jnp.bfloat16),
        mesh=mesh,
        scratch_types=[
            pltpu.VMEM((nf, CH), jnp.int32),
            pltpu.VMEM((CH, C), jnp.bfloat16),
            pltpu.SemaphoreType.DMA,
            pltpu.SemaphoreType.DMA,
        ],
        compiler_params=pltpu.CompilerParams(use_tc_tiling_on_sc=False),
    )
    def body(*refs):
        idxs = refs[:nf]
        tabs = refs[nf:nf + len(tables)]
        if partial_in is None:
            out_hbm, idxb, acc, sem, sem_i = refs[nf + len(tables):]
            part_hbm = None
        else:
            part_hbm, out_hbm, idxb, acc, sem, sem_i = refs[nf + len(tables):]
        wid = lax.axis_index("s") * NC + lax.axis_index("c")

        def do_chunk(cidx, carry):
            row = wid * CHUNKS + cidx
            ih = [pltpu.async_copy(idxs[f].at[row], idxb.at[f], sem_i)
                  for f in range(nf)]
            if partial_in is None:
                zv = jnp.zeros((2 * L,), jnp.bfloat16)

                def zbod(r, _):
                    for cc in range(C // (2 * L)):
                        acc[r, pl.ds(cc * 2 * L, 2 * L)] = zv
                    return 0
                lax.fori_loop(0, CH, zbod, 0)
            else:
                pltpu.sync_copy(part_hbm.at[pl.ds(row * CH, CH)], acc)
            for h in ih:
                h.wait()

            handles = []
            for f in range(nf):
                ti, mult, off = feats_spec[f]
                tab = tabs[ti]
                if (mult, off) != (1, 0):
                    for k in range(CH // L):
                        sl = pl.ds(k * L, L)
                        idxb[f, sl] = idxb[f, sl] * mult + off
                for j in range(CH // GSL):
                    handles.append(pltpu.async_copy(
                        tab.at[idxb.at[f, pl.ds(j * GSL, GSL)]],
                        acc.at[pl.ds(j * GSL, GSL)], sem, add=True))
            for h in handles:
                h.wait()

            pltpu.sync_copy(acc, out_hbm.at[pl.ds(row * CH, CH)])
            return carry

        lax.fori_loop(0, CHUNKS, do_chunk, 0)

    if partial_in is None:
        return body(*idx_list, *tables)
    return body(*idx_list, *tables, partial_in)


def _finish(partial, featp, wf, bp):
    R = 4096

    def fbody(p_ref, f_ref, w_ref, b_ref, o_ref):
        x = (p_ref[...].astype(jnp.float32)
             + jnp.dot(f_ref[...], w_ref[...],
                       preferred_element_type=jnp.float32)
             + b_ref[...])
        col = lax.broadcasted_iota(jnp.int32, (R, C), 1)
        valid = col < CLS
        xm = jnp.where(valid, x, -jnp.inf)
        m = jnp.max(xm, axis=1, keepdims=True)
        e = jnp.where(valid, jnp.exp(x - m), 0.0)
        lse = jnp.log(jnp.sum(e, axis=1, keepdims=True)) + m
        o_ref[...] = (x - lse)[:, :CLS]

    return pl.pallas_call(
        fbody,
        grid=(N // R,),
        in_specs=[pl.BlockSpec((R, C), lambda i: (i, 0)),
                  pl.BlockSpec((R, 15), lambda i: (i, 0)),
                  pl.BlockSpec((15, C), lambda i: (0, 0)),
                  pl.BlockSpec((1, C), lambda i: (0, 0))],
        out_specs=pl.BlockSpec((R, CLS), lambda i: (i, 0)),
        out_shape=jax.ShapeDtypeStruct((N, CLS), jnp.float32),
    )(partial, featp, wf, bp)


def kernel(words, words_suf2, words_suf3, words_pref2, words_pref3, words_feat, prev_words, prev_words_suf2, prev_words_suf3, prev_words_pref2, prev_words_pref3, prev_words_feat, prev_prev_words, prev_prev_words_suf2, prev_prev_words_suf3, prev_prev_words_pref2, prev_prev_words_pref3, prev_prev_words_feat, next_words, next_words_suf2, next_words_suf3, next_words_pref2, next_words_pref3, next_words_feat, next_next_words, next_next_words_suf2, next_next_words_suf3, next_next_words_pref2, next_next_words_pref3, next_next_words_feat, prev_pos, prev_prev_pos, W_word, W_pos, W_suf2, W_suf3, W_pref2, W_pref3, W_lin, b_lin):
    def seg_rhs(off, width):
        rs = jnp.stack([W_lin[s * SEG + off: s * SEG + off + width, :]
                        for s in range(NSLOT)], axis=1)
        rs = jnp.pad(rs, ((0, 0), (0, 0), (0, C - CLS)))
        return rs.reshape(width, NSLOT * C)

    rhs_w = seg_rhs(0, WORD_D)
    rhs_s2 = seg_rhs(WORD_D, OTHER_D)
    rhs_s3 = seg_rhs(WORD_D + OTHER_D, OTHER_D)
    rhs_p2 = seg_rhs(WORD_D + 2 * OTHER_D, OTHER_D)
    rhs_p3 = seg_rhs(WORD_D + 3 * OTHER_D, OTHER_D)
    rhs_pos = jnp.stack([W_lin[NSLOT * SEG: NSLOT * SEG + OTHER_D],
                         W_lin[NSLOT * SEG + OTHER_D:]], axis=1)
    rhs_pos = jnp.pad(rhs_pos, ((0, 0), (0, 0), (0, C - CLS)))
    rhs_pos = rhs_pos.reshape(OTHER_D, 2 * C)
    wf = jnp.stack([W_lin[s * SEG + WORD_D + 4 * OTHER_D: (s + 1) * SEG]
                    for s in range(NSLOT)], axis=0).reshape(15, CLS)
    wf = jnp.pad(wf, ((0, 0), (0, C - CLS)))
    bp = jnp.pad(b_lin, (0, C - CLS)).reshape(1, C)

    sm = _fuse_small(W_suf2, W_suf3, W_pref2, W_pref3,
                     jnp.pad(W_pos, ((0, 3), (0, 0))),
                     rhs_s2, rhs_s3, rhs_p2, rhs_p3, rhs_pos)
    t_s2 = sm[0].reshape(SUF2_V * NSLOT, C)
    t_s3 = sm[1].reshape(SUF3_V * NSLOT, C)
    t_p2 = sm[2].reshape(PREF2_V * NSLOT, C)
    t_p3 = sm[3].reshape(PREF3_V * NSLOT, C)
    t_pos = sm[4][:POS_V].reshape(POS_V * 2, C)

    wordsets = [
        (words, words_suf2, words_suf3, words_pref2, words_pref3),
        (prev_words, prev_words_suf2, prev_words_suf3, prev_words_pref2, prev_words_pref3),
        (prev_prev_words, prev_prev_words_suf2, prev_prev_words_suf3, prev_prev_words_pref2, prev_prev_words_pref3),
        (next_words, next_words_suf2, next_words_suf3, next_words_pref2, next_words_pref3),
        (next_next_words, next_next_words_suf2, next_next_words_suf3, next_next_words_pref2, next_next_words_pref3),
    ]
    featp = jnp.concatenate(
        [words_feat.reshape(N, 3), prev_words_feat.reshape(N, 3),
         prev_prev_words_feat.reshape(N, 3), next_words_feat.reshape(N, 3),
         next_next_words_feat.reshape(N, 3)], axis=1)

    idx_small = [a.reshape(N // CH, CH) for tup in wordsets for a in tup[1:]]
    idx_small += [prev_pos.reshape(N // CH, CH),
                  prev_prev_pos.reshape(N // CH, CH)]
    idx_word = [tup[0].reshape(N // CH, CH) for tup in wordsets]

    smalls = [t_s2, t_s3, t_p2, t_p3, t_pos]
    feats_small = [(i, NSLOT, s) for s in range(NSLOT) for i in range(4)]
    feats_small += [(4, 2, 0), (4, 2, 1)]
    partial1 = _sc_gather_sum(idx_small, smalls, feats_small)

    t_word = _fuse_word(W_word, rhs_w).reshape(WORD_V * NSLOT, C)
    feats_word = [(0, NSLOT, s) for s in range(NSLOT)]
    partial = _sc_gather_sum(idx_word, [t_word], feats_word,
                             partial_in=partial1)

    return _finish(partial, featp, wf, bp)
